# baseline (speedup 1.0000x reference)
import functools

import jax
import jax.numpy as jnp
from jax import lax
from jax.experimental import pallas as pl
from jax.experimental.pallas import tpu as pltpu
from jax.experimental.pallas import tpu_sc as plsc

NUM_EMB = 103
EMB_DIM = 32
BATCH = 1024
SEQ = 50
MAX_PAD = 20

B_TOTAL = BATCH * SEQ * MAX_PAD
NUM_CORES = 2
NUM_SUBCORES = 16
NUM_WORKERS = NUM_CORES * NUM_SUBCORES
ROWS_PER_W = B_TOTAL // NUM_WORKERS
CHUNK = 1280
NCHUNK = ROWS_PER_W // CHUNK
GROUPS = CHUNK // 16


@functools.partial(
    pl.kernel,
    out_type=jax.ShapeDtypeStruct((B_TOTAL * EMB_DIM,), jnp.float32),
    mesh=plsc.VectorSubcoreMesh(core_axis_name="c", subcore_axis_name="s"),
    scratch_types=[
        pltpu.VMEM((NUM_EMB * EMB_DIM,), jnp.float32),
        pltpu.VMEM((ROWS_PER_W,), jnp.int32),
        pltpu.VMEM((CHUNK * EMB_DIM,), jnp.float32),
        pltpu.VMEM((CHUNK * EMB_DIM,), jnp.float32),
        pltpu.SemaphoreType.DMA,
        pltpu.SemaphoreType.DMA,
    ],
    compiler_params=pltpu.CompilerParams(use_tc_tiling_on_sc=False,
                                         needs_layout_passes=False),
)
def _gather_rows(idx_hbm, table_hbm, out_hbm, table_v, idx_v, rows0, rows1,
                 so0, so1):
    wid = lax.axis_index("s") * NUM_CORES + lax.axis_index("c")
    base = wid * (ROWS_PER_W * EMB_DIM)
    rows = (rows0, rows1)
    so = (so0, so1)

    pltpu.sync_copy(table_hbm, table_v)
    pltpu.sync_copy(idx_hbm.at[wid], idx_v)

    lane = lax.iota(jnp.int32, 16)
    obase0 = lane * EMB_DIM

    def fill_chunk(buf, g):
        def group_body(gr, carry):
            vidx = idx_v[pl.ds(g * CHUNK + gr * 16, 16)]
            vbase = vidx * EMB_DIM
            obase = obase0 + gr * (16 * EMB_DIM)
            for c in range(EMB_DIM):
                col = plsc.load_gather(table_v, [vbase + c])
                plsc.store_scatter(buf, [obase + c], col)
            return carry

        lax.fori_loop(0, GROUPS, group_body, 0)

    def flush(b, g):
        pltpu.async_copy(
            rows[b],
            out_hbm.at[pl.ds(base + g * (CHUNK * EMB_DIM), CHUNK * EMB_DIM)],
            so[b])

    def wait_flush(b, g):
        pltpu.make_async_copy(
            rows[b],
            out_hbm.at[pl.ds(base + g * (CHUNK * EMB_DIM), CHUNK * EMB_DIM)],
            so[b]).wait()

    for g in range(NCHUNK):
        b = g % 2
        if g >= 2:
            wait_flush(b, g - 2)
        fill_chunk(rows[b], g)
        flush(b, g)

    wait_flush((NCHUNK - 2) % 2, NCHUNK - 2)
    wait_flush((NCHUNK - 1) % 2, NCHUNK - 1)


def kernel(indices, table):
    B, S, P = indices.shape
    idx = indices.reshape(NUM_WORKERS, ROWS_PER_W).astype(jnp.int32)
    table_flat = table.astype(jnp.float32).reshape(-1)
    out = _gather_rows(idx, table_flat)
    return out.reshape(B, S, P * table.shape[1])


# speedup vs baseline: 4.0521x; 4.0521x over previous
import functools

import jax
import jax.numpy as jnp
from jax import lax
from jax.experimental import pallas as pl
from jax.experimental.pallas import tpu as pltpu
from jax.experimental.pallas import tpu_sc as plsc

NUM_EMB = 103
EMB_DIM = 32
BATCH = 1024
SEQ = 50
MAX_PAD = 20

B_TOTAL = BATCH * SEQ * MAX_PAD
NUM_CORES = 2
NUM_SUBCORES = 16
NUM_WORKERS = NUM_CORES * NUM_SUBCORES
ROWS_PER_W = B_TOTAL // NUM_WORKERS
CHUNK = 800
NCHUNK = ROWS_PER_W // CHUNK
GROUPS = CHUNK // 16
CWORDS = CHUNK * EMB_DIM


@functools.partial(
    pl.kernel,
    out_type=jax.ShapeDtypeStruct((B_TOTAL * EMB_DIM,), jnp.float32),
    mesh=plsc.VectorSubcoreMesh(core_axis_name="c", subcore_axis_name="s"),
    scratch_types=[
        pltpu.VMEM((NUM_EMB * EMB_DIM,), jnp.float32),
        pltpu.VMEM((ROWS_PER_W,), jnp.int32),
        pltpu.VMEM((CWORDS,), jnp.float32),
        pltpu.VMEM((CWORDS,), jnp.float32),
        pltpu.SemaphoreType.DMA,
        pltpu.SemaphoreType.DMA,
    ],
    compiler_params=pltpu.CompilerParams(use_tc_tiling_on_sc=False,
                                         needs_layout_passes=False),
)
def _gather_rows(idx_hbm, table_hbm, out_hbm, table_v, idx_v, rows0, rows1,
                 so0, so1):
    wid = lax.axis_index("s") * NUM_CORES + lax.axis_index("c")
    base = wid * (ROWS_PER_W * EMB_DIM)
    rows = (rows0, rows1)
    so = (so0, so1)

    pltpu.sync_copy(table_hbm, table_v)
    pltpu.sync_copy(idx_hbm.at[wid], idx_v)

    lane = lax.iota(jnp.int32, 16)
    lane32 = lane * EMB_DIM

    def fill_chunk(buf, g):
        @plsc.parallel_loop(0, GROUPS, 1, unroll=2)
        def group_body(u):
            vidx = idx_v[pl.ds(g * CHUNK + u * 16, 16)]
            vbase = vidx * EMB_DIM
            sbase = lane32 + u * (16 * EMB_DIM)
            for c in range(EMB_DIM):
                offv = (lane + c) & (EMB_DIM - 1)
                col = plsc.load_gather(table_v, [vbase + offv])
                plsc.store_scatter(buf, [sbase + offv], col)

    def flush(b, g):
        pltpu.async_copy(rows[b], out_hbm.at[pl.ds(base + g * CWORDS, CWORDS)],
                         so[b])

    def wait_flush(b, g):
        pltpu.make_async_copy(rows[b],
                              out_hbm.at[pl.ds(base + g * CWORDS, CWORDS)],
                              so[b]).wait()

    @pl.loop(0, NCHUNK, step=2)
    def chunk_pair(g):
        for b in range(2):
            @pl.when(g + b >= 2)
            def _():
                wait_flush(b, g + b - 2)

            fill_chunk(rows[b], g + b)
            flush(b, g + b)

    wait_flush(0, NCHUNK - 2)
    wait_flush(1, NCHUNK - 1)


def kernel(indices, table):
    B, S, P = indices.shape
    idx = indices.reshape(NUM_WORKERS, ROWS_PER_W).astype(jnp.int32)
    table_flat = table.astype(jnp.float32).reshape(-1)
    out = _gather_rows(idx, table_flat)
    return out.reshape(B, S, P * table.shape[1])
